# Initial kernel scaffold; baseline (speedup 1.0000x reference)
import jax
import jax.numpy as jnp
from jax.experimental import pallas as pl


def kernel(output, mask, ind, target):
    raise NotImplementedError("write your pallas kernel here")


# speedup vs baseline: 3.1072x; 3.1072x over previous
import functools

import jax
import jax.numpy as jnp
from jax import lax
from jax.experimental import pallas as pl
from jax.experimental.pallas import tpu as pltpu
from jax.experimental.pallas import tpu_sc as plsc

B, C, H, W = 16, 34, 128, 128
K = 32
HW = H * W
L = 16
NC, NS = 2, 16
NW = NC * NS
PAIRS = B * K
PPT = PAIRS // NW
EPT = PPT * C
ROWS = B * C * HW // L

_CHUNKS = [(0, 128), (128, 128), (256, 128), (384, 128), (512, 32)]


@functools.partial(
    pl.kernel,
    out_type=jax.ShapeDtypeStruct((NC, 2 * L), jnp.float32),
    mesh=plsc.VectorSubcoreMesh(
        core_axis_name="c", subcore_axis_name="s", num_cores=NC, num_subcores=NS
    ),
    compiler_params=pltpu.CompilerParams(
        needs_layout_passes=False, use_tc_tiling_on_sc=False
    ),
    scratch_types=[
        pltpu.VMEM((PPT,), jnp.int32),
        pltpu.VMEM((EPT,), jnp.int32),
        pltpu.VMEM((EPT, L), jnp.float32),
        pltpu.VMEM((EPT,), jnp.float32),
        pltpu.VMEM((EPT,), jnp.float32),
        pltpu.VMEM((2 * L,), jnp.float32),
        pltpu.VMEM((NS, 2 * L), jnp.float32),
        pltpu.VMEM_SHARED((NS, 2 * L), jnp.float32),
        pltpu.SemaphoreType.DMA,
    ],
)
def _wl1_sc(table, maskf, indf, targf, out, ind_v, idx_v, rows_v, mask_v,
            targ_v, parts_v, allp_v, shared, sem):
    cid = lax.axis_index("c")
    sid = lax.axis_index("s")
    wid = cid * NS + sid
    b = wid // (K // PPT)
    lane = lax.broadcasted_iota(jnp.int32, (L,), 0)

    pltpu.sync_copy(indf.at[pl.ds(wid * PPT, PPT)], ind_v)
    iv = ind_v[...]
    rem = jnp.bitwise_and(iv, L - 1)

    for c in range(C):
        base = (b * C + c) * HW
        idx_v[pl.ds(c * L, L)] = lax.shift_right_logical(iv + base, 4)

    copies = [
        pltpu.async_copy(table.at[idx_v.at[pl.ds(off, n)]],
                         rows_v.at[pl.ds(off, n)], sem)
        for off, n in _CHUNKS
    ]
    pltpu.sync_copy(maskf.at[pl.ds(wid * EPT, EPT)], mask_v)
    pltpu.sync_copy(targf.at[pl.ds(wid * EPT, EPT)], targ_v)
    for cp in copies:
        cp.wait()

    acc = jnp.zeros((L,), jnp.float32)
    msum = jnp.zeros((L,), jnp.float32)
    for c in range(C):
        p = plsc.load_gather(rows_v, [c * L + lane, rem])
        pos = lane * C + c
        m = plsc.load_gather(mask_v, [pos])
        t = plsc.load_gather(targ_v, [pos])
        acc = acc + jnp.abs(p * m - t * m)
    for i in range(C):
        msum = msum + mask_v[pl.ds(i * L, L)]

    parts_v[pl.ds(0, L)] = acc
    parts_v[pl.ds(L, L)] = msum
    pltpu.sync_copy(parts_v, shared.at[sid])
    plsc.subcore_barrier()

    @pl.when(sid == 0)
    def _():
        pltpu.sync_copy(shared, allp_v)
        a = jnp.zeros((L,), jnp.float32)
        m2 = jnp.zeros((L,), jnp.float32)
        for r in range(NS):
            a = a + allp_v[r, pl.ds(0, L)]
            m2 = m2 + allp_v[r, pl.ds(L, L)]
        parts_v[pl.ds(0, L)] = a
        parts_v[pl.ds(L, L)] = m2
        pltpu.sync_copy(parts_v, out.at[cid])


def kernel(output, mask, ind, target):
    table = output.reshape(ROWS, L)
    maskf = mask.astype(jnp.float32).reshape(-1)
    targf = target.astype(jnp.float32).reshape(-1)
    indf = ind.reshape(-1).astype(jnp.int32)
    parts = _wl1_sc(table, maskf, indf, targf)
    l1 = jnp.sum(parts[:, :L])
    msum = jnp.sum(parts[:, L:])
    return l1 / (msum + 1e-4)


# speedup vs baseline: 3.1301x; 1.0074x over previous
import functools

import jax
import jax.numpy as jnp
from jax import lax
from jax.experimental import pallas as pl
from jax.experimental.pallas import tpu as pltpu
from jax.experimental.pallas import tpu_sc as plsc

B, C, H, W = 16, 34, 128, 128
K = 32
HW = H * W
L = 16
NC, NS = 2, 16
NW = NC * NS
PAIRS = B * K
PPT = PAIRS // NW
EPT = PPT * C
ROWS = B * C * HW // L

_CHUNKS = [(0, 128), (128, 128), (256, 128), (384, 128), (512, 32)]


@functools.partial(
    pl.kernel,
    out_type=jax.ShapeDtypeStruct((NC, 2 * L), jnp.float32),
    mesh=plsc.VectorSubcoreMesh(
        core_axis_name="c", subcore_axis_name="s", num_cores=NC, num_subcores=NS
    ),
    compiler_params=pltpu.CompilerParams(
        needs_layout_passes=False, use_tc_tiling_on_sc=False
    ),
    scratch_types=[
        pltpu.VMEM((PPT,), jnp.int32),
        pltpu.VMEM((EPT,), jnp.int32),
        pltpu.VMEM((EPT, L), jnp.float32),
        pltpu.VMEM((PPT, C), jnp.float32),
        pltpu.VMEM((PPT, C), jnp.float32),
        pltpu.VMEM((2 * L,), jnp.float32),
        pltpu.VMEM((NS, 2 * L), jnp.float32),
        pltpu.VMEM_SHARED((NS, 2 * L), jnp.float32),
        pltpu.SemaphoreType.DMA,
    ],
)
def _wl1_sc(table, mask3, ind2, targ3, out, ind_v, idx_v, rows_v, mask_v,
            targ_v, parts_v, allp_v, shared, sem):
    cid = lax.axis_index("c")
    sid = lax.axis_index("s")
    wid = cid * NS + sid
    b = wid // (K // PPT)
    k0 = (wid % (K // PPT)) * PPT
    lane = lax.broadcasted_iota(jnp.int32, (L,), 0)

    pltpu.sync_copy(ind2.at[b, pl.ds(k0, PPT)], ind_v)
    iv = ind_v[...]
    rem = jnp.bitwise_and(iv, L - 1)

    for c in range(C):
        base = (b * C + c) * HW
        idx_v[pl.ds(c * L, L)] = lax.shift_right_logical(iv + base, 4)

    copies = [
        pltpu.async_copy(table.at[idx_v.at[pl.ds(off, n)]],
                         rows_v.at[pl.ds(off, n)], sem)
        for off, n in _CHUNKS
    ]
    pltpu.sync_copy(mask3.at[b, pl.ds(k0, PPT)], mask_v)
    pltpu.sync_copy(targ3.at[b, pl.ds(k0, PPT)], targ_v)
    for cp in copies:
        cp.wait()

    acc = jnp.zeros((L,), jnp.float32)
    msum = jnp.zeros((L,), jnp.float32)
    for c in range(C):
        cs = jnp.full((L,), c, jnp.int32)
        p = plsc.load_gather(rows_v, [c * L + lane, rem])
        m = plsc.load_gather(mask_v, [lane, cs])
        t = plsc.load_gather(targ_v, [lane, cs])
        acc = acc + jnp.abs(p * m - t * m)
        msum = msum + m

    parts_v[pl.ds(0, L)] = acc
    parts_v[pl.ds(L, L)] = msum
    pltpu.sync_copy(parts_v, shared.at[sid])
    plsc.subcore_barrier()

    @pl.when(sid == 0)
    def _():
        pltpu.sync_copy(shared, allp_v)
        a = jnp.zeros((L,), jnp.float32)
        m2 = jnp.zeros((L,), jnp.float32)
        for r in range(NS):
            a = a + allp_v[r, pl.ds(0, L)]
            m2 = m2 + allp_v[r, pl.ds(L, L)]
        parts_v[pl.ds(0, L)] = a
        parts_v[pl.ds(L, L)] = m2
        pltpu.sync_copy(parts_v, out.at[cid])


def kernel(output, mask, ind, target):
    table = output.reshape(ROWS, L)
    parts = _wl1_sc(table, mask.astype(jnp.float32), ind.astype(jnp.int32),
                    target.astype(jnp.float32))
    l1 = jnp.sum(parts[:, :L])
    msum = jnp.sum(parts[:, L:])
    return l1 / (msum + 1e-4)
